# baseline (speedup 1.0000x reference)
import functools

import jax
import jax.numpy as jnp
from jax import lax
from jax.experimental import pallas as pl
from jax.experimental.pallas import tpu as pltpu
from jax.experimental.pallas import tpu_sc as plsc

H = 128
_NC = 2
_NS = 16
_NW = _NC * _NS


def _pre_body(x_ref, w1a_ref, w1b_ref, b1_ref, xa_ref, xb_ref):
    x = x_ref[...]
    xa = jnp.dot(x, w1a_ref[...], preferred_element_type=jnp.float32)
    xb = (jnp.dot(x, w1b_ref[...], preferred_element_type=jnp.float32)
          + b1_ref[...])
    xa_ref[...] = xa
    xb_ref[...] = xb


def _pre(x, w1a, w1b, b1, bn=1000):
    n = x.shape[0]
    return pl.pallas_call(
        _pre_body,
        grid=(n // bn,),
        in_specs=[
            pl.BlockSpec((bn, H), lambda i: (i, 0)),
            pl.BlockSpec((H, H), lambda i: (0, 0)),
            pl.BlockSpec((H, H), lambda i: (0, 0)),
            pl.BlockSpec((1, H), lambda i: (0, 0)),
        ],
        out_specs=[pl.BlockSpec((bn, H), lambda i: (i, 0))] * 2,
        out_shape=[jax.ShapeDtypeStruct((n, H), jnp.float32)] * 2,
    )(x, w1a, w1b, b1)


_NBUF = 5


def _gather(xa_bf, xb_bf, src, dst):
    e = src.shape[0]
    per_w = e // _NW
    c_sz = 80 if per_w % (80 * _NBUF) == 0 else 40
    n_groups = per_w // (c_sz * _NBUF)
    mesh = plsc.VectorSubcoreMesh(core_axis_name="c", subcore_axis_name="s")

    scratch = ([pltpu.VMEM((c_sz,), jnp.int32)] * (2 * _NBUF)
               + [pltpu.VMEM((c_sz, H), jnp.float32)] * (2 * _NBUF)
               + [pltpu.SemaphoreType.DMA] * (3 * _NBUF))

    @functools.partial(
        pl.kernel, mesh=mesh,
        out_type=[jax.ShapeDtypeStruct((e, H), jnp.float32)] * 2,
        scratch_types=scratch,
    )
    def k(xa_hbm, xb_hbm, src_hbm, dst_hbm, xs_hbm, xd_hbm, *scr):
        idx_s = scr[0:_NBUF]
        idx_d = scr[_NBUF:2 * _NBUF]
        rows_a = scr[2 * _NBUF:3 * _NBUF]
        rows_b = scr[3 * _NBUF:4 * _NBUF]
        sem_i = scr[4 * _NBUF:5 * _NBUF]
        sem_g = scr[5 * _NBUF:6 * _NBUF]
        sem_w = scr[6 * _NBUF:7 * _NBUF]
        wid = lax.axis_index("s") * _NC + lax.axis_index("c")
        base = wid * per_w

        def body(j, carry):
            offs = [base + (j * _NBUF + b) * c_sz for b in range(_NBUF)]
            for b in range(_NBUF):
                off = offs[b]

                @pl.when(j > 0)
                def _(off=off, b=b):
                    pltpu.make_async_copy(
                        rows_a[b], xs_hbm.at[pl.ds(off, c_sz)], sem_w[b]).wait()
                    pltpu.make_async_copy(
                        rows_b[b], xd_hbm.at[pl.ds(off, c_sz)], sem_w[b]).wait()

                pltpu.async_copy(src_hbm.at[pl.ds(off, c_sz)], idx_s[b], sem_i[b])
                pltpu.async_copy(dst_hbm.at[pl.ds(off, c_sz)], idx_d[b], sem_i[b])
            gathers = []
            for b in range(_NBUF):
                off = offs[b]
                pltpu.make_async_copy(
                    src_hbm.at[pl.ds(off, c_sz)], idx_s[b], sem_i[b]).wait()
                pltpu.make_async_copy(
                    dst_hbm.at[pl.ds(off, c_sz)], idx_d[b], sem_i[b]).wait()
                ga = pltpu.async_copy(xa_hbm.at[idx_s[b]], rows_a[b], sem_g[b])
                gb = pltpu.async_copy(xb_hbm.at[idx_d[b]], rows_b[b], sem_g[b])
                gathers.append((ga, gb))
            for b in range(_NBUF):
                off = offs[b]
                ga, gb = gathers[b]
                ga.wait()
                gb.wait()
                pltpu.async_copy(rows_a[b], xs_hbm.at[pl.ds(off, c_sz)], sem_w[b])
                pltpu.async_copy(rows_b[b], xd_hbm.at[pl.ds(off, c_sz)], sem_w[b])
            return carry

        lax.fori_loop(0, n_groups, body, 0)
        for b in range(_NBUF):
            pltpu.make_async_copy(
                rows_a[b], xs_hbm.at[pl.ds(base, c_sz)], sem_w[b]).wait()
            pltpu.make_async_copy(
                rows_b[b], xd_hbm.at[pl.ds(base, c_sz)], sem_w[b]).wait()

    return k(xa_bf, xb_bf, src, dst)


def _edge_body_alias(xs_ref, xd_ref, ea_ref, w1c_ref, w2_ref, b2_ref, ws_ref,
                     bs_ref, dprev_ref, d_ref, m_ref):
    del dprev_ref
    _edge_body(xs_ref, xd_ref, ea_ref, w1c_ref, w2_ref, b2_ref, ws_ref,
               bs_ref, d_ref, m_ref)


def _edge_body(xs_ref, xd_ref, ea_ref, w1c_ref, w2_ref, b2_ref, ws_ref,
               bs_ref, d_ref, m_ref):
    ea = ea_ref[...]
    pre = (xs_ref[...] + xd_ref[...]
           + jnp.dot(ea.astype(jnp.bfloat16),
                     w1c_ref[...].astype(jnp.bfloat16),
                     preferred_element_type=jnp.float32))
    h = jnp.maximum(pre, 0.0)
    msg = jnp.maximum(
        jnp.dot(h.astype(jnp.bfloat16), w2_ref[...].astype(jnp.bfloat16),
                preferred_element_type=jnp.float32)
        + b2_ref[...], 0.0)
    d_ref[...] = ea + msg
    gate = jax.nn.sigmoid(
        jnp.sum(msg * ws_ref[...], axis=-1, keepdims=True) + bs_ref[...])
    m_ref[...] = msg * gate


def _edge(xs, xd, ea, w1c, w2, b2, ws, bs, d_prev, slab, e_tot, be=3200):
    e_s = ea.shape[0]
    nblk = e_s // be
    base_blk = slab * nblk
    blk = pl.BlockSpec((be, H), lambda i: (i, 0))
    pblk = pl.BlockSpec((be, H), lambda i: (i, 0))
    wblk = pl.BlockSpec((H, H), lambda i: (0, 0))
    rowblk = pl.BlockSpec((1, H), lambda i: (0, 0))
    in_specs = [pblk, pblk, blk, wblk, wblk, rowblk, rowblk,
                pl.BlockSpec((1, 1), lambda i: (0, 0))]
    inputs = [xs, xd, ea, w1c, w2, b2, ws, bs]
    io_alias = {}
    body = _edge_body
    if d_prev is not None:
        in_specs.append(pl.BlockSpec(memory_space=pl.ANY))
        inputs.append(d_prev)
        io_alias = {8: 0}
        body = _edge_body_alias
    return pl.pallas_call(
        body,
        grid=(nblk,),
        in_specs=in_specs,
        out_specs=[pl.BlockSpec((be, H), lambda i: (base_blk + i, 0)), blk],
        out_shape=[jax.ShapeDtypeStruct((e_tot, H), jnp.float32),
                   jax.ShapeDtypeStruct((e_s, H), jnp.float32)],
        input_output_aliases=io_alias,
    )(*inputs)


def _scatter(m, dst, zeros):
    e = m.shape[0]
    n = zeros.shape[1]
    per_w = e // _NW
    c_sz = 40
    rows_per_s = n // _NS
    mesh = plsc.VectorSubcoreMesh(core_axis_name="c", subcore_axis_name="s")

    scratch = ([pltpu.VMEM((c_sz,), jnp.int32)] * _NBUF
               + [pltpu.VMEM((c_sz, H), jnp.float32)] * _NBUF
               + [pltpu.VMEM_SHARED((n, H), jnp.float32)]
               + [pltpu.SemaphoreType.DMA] * (2 * _NBUF))

    @functools.partial(
        pl.kernel, mesh=mesh,
        out_type=jax.ShapeDtypeStruct((_NC, n, H), jnp.float32),
        scratch_types=scratch,
    )
    def k(m_hbm, dst_hbm, z_hbm, out_hbm, *scr):
        idx_v = scr[0:_NBUF]
        rows_v = scr[_NBUF:2 * _NBUF]
        acc = scr[2 * _NBUF]
        sem_l = scr[2 * _NBUF + 1:3 * _NBUF + 1]
        sem_s = scr[3 * _NBUF + 1:4 * _NBUF + 1]
        c = lax.axis_index("c")
        s = lax.axis_index("s")
        wid = s * _NC + c
        pltpu.sync_copy(z_hbm.at[c, pl.ds(s * rows_per_s, rows_per_s)],
                        acc.at[pl.ds(s * rows_per_s, rows_per_s)])
        plsc.subcore_barrier()
        base = wid * per_w

        def body(j, carry):
            offs = [base + (j * _NBUF + b) * c_sz for b in range(_NBUF)]
            for b in range(_NBUF):
                off = offs[b]

                @pl.when(j > 0)
                def _(b=b):
                    pltpu.make_async_copy(
                        rows_v[b], acc.at[idx_v[b]], sem_s[b]).wait()

                pltpu.async_copy(dst_hbm.at[pl.ds(off, c_sz)], idx_v[b], sem_l[b])
                pltpu.async_copy(m_hbm.at[pl.ds(off, c_sz)], rows_v[b], sem_l[b])
            for b in range(_NBUF):
                off = offs[b]
                pltpu.make_async_copy(
                    dst_hbm.at[pl.ds(off, c_sz)], idx_v[b], sem_l[b]).wait()
                pltpu.make_async_copy(
                    m_hbm.at[pl.ds(off, c_sz)], rows_v[b], sem_l[b]).wait()
                pltpu.async_copy(rows_v[b], acc.at[idx_v[b]], sem_s[b], add=True)
            return carry

        lax.fori_loop(0, per_w // (c_sz * _NBUF), body, 0)
        for b in range(_NBUF):
            pltpu.make_async_copy(rows_v[b], acc.at[idx_v[b]], sem_s[b]).wait()
        plsc.subcore_barrier()
        pltpu.sync_copy(acc.at[pl.ds(s * rows_per_s, rows_per_s)],
                        out_hbm.at[c, pl.ds(s * rows_per_s, rows_per_s)])

    return k(m, dst, zeros)


def _node_body(*refs):
    part_refs = refs[:-6]
    x_ref, wu1_ref, bu1_ref, wu2_ref, bu2_ref, o_ref = refs[-6:]
    x = x_ref[...]
    inp = x
    for p_ref in part_refs:
        p = p_ref[...]
        inp = inp + p[0] + p[1]
    h = jnp.maximum(
        jnp.dot(inp, wu1_ref[...], preferred_element_type=jnp.float32)
        + bu1_ref[...], 0.0)
    o_ref[...] = (jnp.dot(h, wu2_ref[...], preferred_element_type=jnp.float32)
                  + bu2_ref[...] + x)


def _node(parts, x, wu1, bu1, wu2, bu2, bn=1000):
    n = x.shape[0]
    pspec = pl.BlockSpec((_NC, bn, H), lambda i: (0, i, 0))
    return pl.pallas_call(
        _node_body,
        grid=(n // bn,),
        in_specs=[pspec] * len(parts) + [
            pl.BlockSpec((bn, H), lambda i: (i, 0)),
            pl.BlockSpec((H, H), lambda i: (0, 0)),
            pl.BlockSpec((1, H), lambda i: (0, 0)),
            pl.BlockSpec((H, H), lambda i: (0, 0)),
            pl.BlockSpec((1, H), lambda i: (0, 0)),
        ],
        out_specs=pl.BlockSpec((bn, H), lambda i: (i, 0)),
        out_shape=jax.ShapeDtypeStruct((n, H), jnp.float32),
    )(*parts, x, wu1, bu1, wu2, bu2)


def kernel(x, edge_index, edge_attr, W_msg1, b_msg1, W_msg2, b_msg2,
           W_soft, b_soft, W_upd1, b_upd1, W_upd2, b_upd2):
    n = x.shape[0]
    src = edge_index[0]
    dst = edge_index[1]
    w1a = W_msg1[:H]
    w1b = W_msg1[H:2 * H]
    w1c = W_msg1[2 * H:]
    b1 = b_msg1.reshape(1, H)
    b2 = b_msg2.reshape(1, H)
    ws = W_soft.reshape(1, H)
    bs = b_soft.reshape(1, 1)

    xa_bf, xb_bf = _pre(x, w1a, w1b, b1)
    n_pad = ((n + _NS * 8 - 1) // (_NS * 8)) * (_NS * 8)
    zeros = jnp.zeros((_NC, n_pad, H), jnp.float32)

    e_tot = edge_attr.shape[0]
    n_slabs = 2
    e_s = e_tot // n_slabs
    d_new = None
    parts = []
    for i in range(n_slabs):
        sl = slice(i * e_s, (i + 1) * e_s)
        xs, xd = _gather(xa_bf, xb_bf, src[sl], dst[sl])
        d_new, m = _edge(xs, xd, edge_attr[sl], w1c, W_msg2, b2, ws, bs,
                         d_new, i, e_tot)
        parts.append(_scatter(m, dst[sl], zeros))

    feat_out = _node(parts, x, W_upd1, b_upd1.reshape(1, H),
                     W_upd2, b_upd2.reshape(1, H))
    return feat_out, d_new


# speedup vs baseline: 1.0106x; 1.0106x over previous
import functools

import jax
import jax.numpy as jnp
from jax import lax
from jax.experimental import pallas as pl
from jax.experimental.pallas import tpu as pltpu
from jax.experimental.pallas import tpu_sc as plsc

H = 128
_NC = 2
_NS = 16
_NW = _NC * _NS


def _pre_body(x_ref, w1a_ref, w1b_ref, b1_ref, xa_ref, xb_ref):
    x = x_ref[...]
    xa = jnp.dot(x, w1a_ref[...], preferred_element_type=jnp.float32)
    xb = (jnp.dot(x, w1b_ref[...], preferred_element_type=jnp.float32)
          + b1_ref[...])
    xa_ref[...] = xa
    xb_ref[...] = xb


def _pre(x, w1a, w1b, b1, bn=1000):
    n = x.shape[0]
    return pl.pallas_call(
        _pre_body,
        grid=(n // bn,),
        in_specs=[
            pl.BlockSpec((bn, H), lambda i: (i, 0)),
            pl.BlockSpec((H, H), lambda i: (0, 0)),
            pl.BlockSpec((H, H), lambda i: (0, 0)),
            pl.BlockSpec((1, H), lambda i: (0, 0)),
        ],
        out_specs=[pl.BlockSpec((bn, H), lambda i: (i, 0))] * 2,
        out_shape=[jax.ShapeDtypeStruct((n, H), jnp.float32)] * 2,
    )(x, w1a, w1b, b1)


_NBUF = 5


def _gather(xa_bf, xb_bf, src, dst):
    e = src.shape[0]
    per_w = e // _NW
    c_sz = 80 if per_w % (80 * _NBUF) == 0 else 40
    n_groups = per_w // (c_sz * _NBUF)
    mesh = plsc.VectorSubcoreMesh(core_axis_name="c", subcore_axis_name="s")

    scratch = ([pltpu.VMEM((c_sz,), jnp.int32)] * (2 * _NBUF)
               + [pltpu.VMEM((c_sz, H), jnp.float32)] * (2 * _NBUF)
               + [pltpu.SemaphoreType.DMA] * (3 * _NBUF))

    @functools.partial(
        pl.kernel, mesh=mesh,
        out_type=[jax.ShapeDtypeStruct((e, H), jnp.float32)] * 2,
        scratch_types=scratch,
    )
    def k(xa_hbm, xb_hbm, src_hbm, dst_hbm, xs_hbm, xd_hbm, *scr):
        idx_s = scr[0:_NBUF]
        idx_d = scr[_NBUF:2 * _NBUF]
        rows_a = scr[2 * _NBUF:3 * _NBUF]
        rows_b = scr[3 * _NBUF:4 * _NBUF]
        sem_i = scr[4 * _NBUF:5 * _NBUF]
        sem_g = scr[5 * _NBUF:6 * _NBUF]
        sem_w = scr[6 * _NBUF:7 * _NBUF]
        wid = lax.axis_index("s") * _NC + lax.axis_index("c")
        base = wid * per_w

        def body(j, carry):
            offs = [base + (j * _NBUF + b) * c_sz for b in range(_NBUF)]
            for b in range(_NBUF):
                off = offs[b]

                @pl.when(j > 0)
                def _(off=off, b=b):
                    pltpu.make_async_copy(
                        rows_a[b], xs_hbm.at[pl.ds(off, c_sz)], sem_w[b]).wait()
                    pltpu.make_async_copy(
                        rows_b[b], xd_hbm.at[pl.ds(off, c_sz)], sem_w[b]).wait()

                pltpu.async_copy(src_hbm.at[pl.ds(off, c_sz)], idx_s[b], sem_i[b])
                pltpu.async_copy(dst_hbm.at[pl.ds(off, c_sz)], idx_d[b], sem_i[b])
            gathers = []
            for b in range(_NBUF):
                off = offs[b]
                pltpu.make_async_copy(
                    src_hbm.at[pl.ds(off, c_sz)], idx_s[b], sem_i[b]).wait()
                pltpu.make_async_copy(
                    dst_hbm.at[pl.ds(off, c_sz)], idx_d[b], sem_i[b]).wait()
                ga = pltpu.async_copy(xa_hbm.at[idx_s[b]], rows_a[b], sem_g[b])
                gb = pltpu.async_copy(xb_hbm.at[idx_d[b]], rows_b[b], sem_g[b])
                gathers.append((ga, gb))
            for b in range(_NBUF):
                off = offs[b]
                ga, gb = gathers[b]
                ga.wait()
                gb.wait()
                pltpu.async_copy(rows_a[b], xs_hbm.at[pl.ds(off, c_sz)], sem_w[b])
                pltpu.async_copy(rows_b[b], xd_hbm.at[pl.ds(off, c_sz)], sem_w[b])
            return carry

        lax.fori_loop(0, n_groups, body, 0)
        for b in range(_NBUF):
            pltpu.make_async_copy(
                rows_a[b], xs_hbm.at[pl.ds(base, c_sz)], sem_w[b]).wait()
            pltpu.make_async_copy(
                rows_b[b], xd_hbm.at[pl.ds(base, c_sz)], sem_w[b]).wait()

    return k(xa_bf, xb_bf, src, dst)


def _edge_body_alias(xs_ref, xd_ref, ea_ref, w1c_ref, w2_ref, b2_ref, ws_ref,
                     bs_ref, dprev_ref, d_ref, m_ref):
    del dprev_ref
    _edge_body(xs_ref, xd_ref, ea_ref, w1c_ref, w2_ref, b2_ref, ws_ref,
               bs_ref, d_ref, m_ref)


def _edge_body(xs_ref, xd_ref, ea_ref, w1c_ref, w2_ref, b2_ref, ws_ref,
               bs_ref, d_ref, m_ref):
    ea = ea_ref[...]
    pre = (xs_ref[...] + xd_ref[...]
           + jnp.dot(ea, w1c_ref[...], preferred_element_type=jnp.float32))
    h = jnp.maximum(pre, 0.0)
    msg = jnp.maximum(
        jnp.dot(h, w2_ref[...], preferred_element_type=jnp.float32)
        + b2_ref[...], 0.0)
    d_ref[...] = ea + msg
    gate = jax.nn.sigmoid(
        jnp.sum(msg * ws_ref[...], axis=-1, keepdims=True) + bs_ref[...])
    m_ref[...] = msg * gate


def _edge(xs, xd, ea, w1c, w2, b2, ws, bs, d_prev, slab, e_tot, be=3200):
    e_s = ea.shape[0]
    nblk = e_s // be
    base_blk = slab * nblk
    blk = pl.BlockSpec((be, H), lambda i: (i, 0))
    pblk = pl.BlockSpec((be, H), lambda i: (i, 0))
    wblk = pl.BlockSpec((H, H), lambda i: (0, 0))
    rowblk = pl.BlockSpec((1, H), lambda i: (0, 0))
    in_specs = [pblk, pblk, blk, wblk, wblk, rowblk, rowblk,
                pl.BlockSpec((1, 1), lambda i: (0, 0))]
    inputs = [xs, xd, ea, w1c, w2, b2, ws, bs]
    io_alias = {}
    body = _edge_body
    if d_prev is not None:
        in_specs.append(pl.BlockSpec(memory_space=pl.ANY))
        inputs.append(d_prev)
        io_alias = {8: 0}
        body = _edge_body_alias
    return pl.pallas_call(
        body,
        grid=(nblk,),
        in_specs=in_specs,
        out_specs=[pl.BlockSpec((be, H), lambda i: (base_blk + i, 0)), blk],
        out_shape=[jax.ShapeDtypeStruct((e_tot, H), jnp.float32),
                   jax.ShapeDtypeStruct((e_s, H), jnp.float32)],
        input_output_aliases=io_alias,
    )(*inputs)


def _scatter(m, dst, zeros):
    e = m.shape[0]
    n = zeros.shape[1]
    per_w = e // _NW
    c_sz = 40
    rows_per_s = n // _NS
    mesh = plsc.VectorSubcoreMesh(core_axis_name="c", subcore_axis_name="s")

    scratch = ([pltpu.VMEM((c_sz,), jnp.int32)] * _NBUF
               + [pltpu.VMEM((c_sz, H), jnp.float32)] * _NBUF
               + [pltpu.VMEM_SHARED((n, H), jnp.float32)]
               + [pltpu.SemaphoreType.DMA] * (2 * _NBUF))

    @functools.partial(
        pl.kernel, mesh=mesh,
        out_type=jax.ShapeDtypeStruct((_NC, n, H), jnp.float32),
        scratch_types=scratch,
    )
    def k(m_hbm, dst_hbm, z_hbm, out_hbm, *scr):
        idx_v = scr[0:_NBUF]
        rows_v = scr[_NBUF:2 * _NBUF]
        acc = scr[2 * _NBUF]
        sem_l = scr[2 * _NBUF + 1:3 * _NBUF + 1]
        sem_s = scr[3 * _NBUF + 1:4 * _NBUF + 1]
        c = lax.axis_index("c")
        s = lax.axis_index("s")
        wid = s * _NC + c
        pltpu.sync_copy(z_hbm.at[c, pl.ds(s * rows_per_s, rows_per_s)],
                        acc.at[pl.ds(s * rows_per_s, rows_per_s)])
        plsc.subcore_barrier()
        base = wid * per_w

        def body(j, carry):
            offs = [base + (j * _NBUF + b) * c_sz for b in range(_NBUF)]
            for b in range(_NBUF):
                off = offs[b]

                @pl.when(j > 0)
                def _(b=b):
                    pltpu.make_async_copy(
                        rows_v[b], acc.at[idx_v[b]], sem_s[b]).wait()

                pltpu.async_copy(dst_hbm.at[pl.ds(off, c_sz)], idx_v[b], sem_l[b])
                pltpu.async_copy(m_hbm.at[pl.ds(off, c_sz)], rows_v[b], sem_l[b])
            for b in range(_NBUF):
                off = offs[b]
                pltpu.make_async_copy(
                    dst_hbm.at[pl.ds(off, c_sz)], idx_v[b], sem_l[b]).wait()
                pltpu.make_async_copy(
                    m_hbm.at[pl.ds(off, c_sz)], rows_v[b], sem_l[b]).wait()
                pltpu.async_copy(rows_v[b], acc.at[idx_v[b]], sem_s[b], add=True)
            return carry

        lax.fori_loop(0, per_w // (c_sz * _NBUF), body, 0)
        for b in range(_NBUF):
            pltpu.make_async_copy(rows_v[b], acc.at[idx_v[b]], sem_s[b]).wait()
        plsc.subcore_barrier()
        pltpu.sync_copy(acc.at[pl.ds(s * rows_per_s, rows_per_s)],
                        out_hbm.at[c, pl.ds(s * rows_per_s, rows_per_s)])

    return k(m, dst, zeros)


def _node_body(*refs):
    part_refs = refs[:-6]
    x_ref, wu1_ref, bu1_ref, wu2_ref, bu2_ref, o_ref = refs[-6:]
    x = x_ref[...]
    inp = x
    for p_ref in part_refs:
        p = p_ref[...]
        inp = inp + p[0] + p[1]
    h = jnp.maximum(
        jnp.dot(inp, wu1_ref[...], preferred_element_type=jnp.float32)
        + bu1_ref[...], 0.0)
    o_ref[...] = (jnp.dot(h, wu2_ref[...], preferred_element_type=jnp.float32)
                  + bu2_ref[...] + x)


def _node(parts, x, wu1, bu1, wu2, bu2, bn=1000):
    n = x.shape[0]
    pspec = pl.BlockSpec((_NC, bn, H), lambda i: (0, i, 0))
    return pl.pallas_call(
        _node_body,
        grid=(n // bn,),
        in_specs=[pspec] * len(parts) + [
            pl.BlockSpec((bn, H), lambda i: (i, 0)),
            pl.BlockSpec((H, H), lambda i: (0, 0)),
            pl.BlockSpec((1, H), lambda i: (0, 0)),
            pl.BlockSpec((H, H), lambda i: (0, 0)),
            pl.BlockSpec((1, H), lambda i: (0, 0)),
        ],
        out_specs=pl.BlockSpec((bn, H), lambda i: (i, 0)),
        out_shape=jax.ShapeDtypeStruct((n, H), jnp.float32),
    )(*parts, x, wu1, bu1, wu2, bu2)


def kernel(x, edge_index, edge_attr, W_msg1, b_msg1, W_msg2, b_msg2,
           W_soft, b_soft, W_upd1, b_upd1, W_upd2, b_upd2):
    n = x.shape[0]
    src = edge_index[0]
    dst = edge_index[1]
    w1a = W_msg1[:H]
    w1b = W_msg1[H:2 * H]
    w1c = W_msg1[2 * H:]
    b1 = b_msg1.reshape(1, H)
    b2 = b_msg2.reshape(1, H)
    ws = W_soft.reshape(1, H)
    bs = b_soft.reshape(1, 1)

    xa_bf, xb_bf = _pre(x, w1a, w1b, b1)
    n_pad = ((n + _NS * 8 - 1) // (_NS * 8)) * (_NS * 8)
    zeros = jnp.zeros((_NC, n_pad, H), jnp.float32)

    e_tot = edge_attr.shape[0]
    n_slabs = 1
    e_s = e_tot // n_slabs
    d_new = None
    parts = []
    for i in range(n_slabs):
        sl = slice(i * e_s, (i + 1) * e_s)
        xs, xd = _gather(xa_bf, xb_bf, src[sl], dst[sl])
        d_new, m = _edge(xs, xd, edge_attr[sl], w1c, W_msg2, b2, ws, bs,
                         d_new, i, e_tot)
        parts.append(_scatter(m, dst[sl], zeros))

    feat_out = _node(parts, x, W_upd1, b_upd1.reshape(1, H),
                     W_upd2, b_upd2.reshape(1, H))
    return feat_out, d_new


# speedup vs baseline: 1.1907x; 1.1782x over previous
import functools

import jax
import jax.numpy as jnp
from jax import lax
from jax.experimental import pallas as pl
from jax.experimental.pallas import tpu as pltpu
from jax.experimental.pallas import tpu_sc as plsc

H = 128
_NC = 2
_NS = 16
_NW = _NC * _NS


def _pre_body(x_ref, w1a_ref, w1b_ref, b1_ref, xa_ref, xb_ref):
    x = x_ref[...]
    xa = jnp.dot(x, w1a_ref[...], preferred_element_type=jnp.float32)
    xb = (jnp.dot(x, w1b_ref[...], preferred_element_type=jnp.float32)
          + b1_ref[...])
    xa_ref[...] = xa
    xb_ref[...] = xb


def _pre(x, w1a, w1b, b1, bn=1000):
    n = x.shape[0]
    return pl.pallas_call(
        _pre_body,
        grid=(n // bn,),
        in_specs=[
            pl.BlockSpec((bn, H), lambda i: (i, 0)),
            pl.BlockSpec((H, H), lambda i: (0, 0)),
            pl.BlockSpec((H, H), lambda i: (0, 0)),
            pl.BlockSpec((1, H), lambda i: (0, 0)),
        ],
        out_specs=[pl.BlockSpec((bn, H), lambda i: (i, 0))] * 2,
        out_shape=[jax.ShapeDtypeStruct((n, H), jnp.float32)] * 2,
    )(x, w1a, w1b, b1)


_NBUF = 5


def _gather(xa_bf, xb_bf, src, dst):
    e = src.shape[0]
    per_w = e // _NW
    c_sz = 80 if per_w % (80 * _NBUF) == 0 else 40
    n_groups = per_w // (c_sz * _NBUF)
    mesh = plsc.VectorSubcoreMesh(core_axis_name="c", subcore_axis_name="s")

    scratch = ([pltpu.VMEM((c_sz,), jnp.int32)] * (2 * _NBUF)
               + [pltpu.VMEM((c_sz, H), jnp.float32)] * (2 * _NBUF)
               + [pltpu.SemaphoreType.DMA] * (3 * _NBUF))

    @functools.partial(
        pl.kernel, mesh=mesh,
        out_type=[jax.ShapeDtypeStruct((e, H), jnp.float32)] * 2,
        scratch_types=scratch,
    )
    def k(xa_hbm, xb_hbm, src_hbm, dst_hbm, xs_hbm, xd_hbm, *scr):
        idx_s = scr[0:_NBUF]
        idx_d = scr[_NBUF:2 * _NBUF]
        rows_a = scr[2 * _NBUF:3 * _NBUF]
        rows_b = scr[3 * _NBUF:4 * _NBUF]
        sem_i = scr[4 * _NBUF:5 * _NBUF]
        sem_g = scr[5 * _NBUF:6 * _NBUF]
        sem_w = scr[6 * _NBUF:7 * _NBUF]
        wid = lax.axis_index("s") * _NC + lax.axis_index("c")
        base = wid * per_w

        def body(j, carry):
            offs = [base + (j * _NBUF + b) * c_sz for b in range(_NBUF)]
            for b in range(_NBUF):
                off = offs[b]

                @pl.when(j > 0)
                def _(off=off, b=b):
                    pltpu.make_async_copy(
                        rows_a[b], xs_hbm.at[pl.ds(off, c_sz)], sem_w[b]).wait()
                    pltpu.make_async_copy(
                        rows_b[b], xd_hbm.at[pl.ds(off, c_sz)], sem_w[b]).wait()

                pltpu.async_copy(src_hbm.at[pl.ds(off, c_sz)], idx_s[b], sem_i[b])
                pltpu.async_copy(dst_hbm.at[pl.ds(off, c_sz)], idx_d[b], sem_i[b])
            gathers = []
            for b in range(_NBUF):
                off = offs[b]
                pltpu.make_async_copy(
                    src_hbm.at[pl.ds(off, c_sz)], idx_s[b], sem_i[b]).wait()
                pltpu.make_async_copy(
                    dst_hbm.at[pl.ds(off, c_sz)], idx_d[b], sem_i[b]).wait()
                ga = pltpu.async_copy(xa_hbm.at[idx_s[b]], rows_a[b], sem_g[b])
                gb = pltpu.async_copy(xb_hbm.at[idx_d[b]], rows_b[b], sem_g[b])
                gathers.append((ga, gb))
            for b in range(_NBUF):
                off = offs[b]
                ga, gb = gathers[b]
                ga.wait()
                gb.wait()
                pltpu.async_copy(rows_a[b], xs_hbm.at[pl.ds(off, c_sz)], sem_w[b])
                pltpu.async_copy(rows_b[b], xd_hbm.at[pl.ds(off, c_sz)], sem_w[b])
            return carry

        lax.fori_loop(0, n_groups, body, 0)
        for b in range(_NBUF):
            pltpu.make_async_copy(
                rows_a[b], xs_hbm.at[pl.ds(base, c_sz)], sem_w[b]).wait()
            pltpu.make_async_copy(
                rows_b[b], xd_hbm.at[pl.ds(base, c_sz)], sem_w[b]).wait()

    return k(xa_bf, xb_bf, src, dst)


def _edge_body_alias(xs_ref, xd_ref, ea_ref, w1c_ref, w2_ref, b2_ref, ws_ref,
                     bs_ref, dprev_ref, d_ref, m_ref):
    del dprev_ref
    _edge_body(xs_ref, xd_ref, ea_ref, w1c_ref, w2_ref, b2_ref, ws_ref,
               bs_ref, d_ref, m_ref)


def _edge_body(xs_ref, xd_ref, ea_ref, w1c_ref, w2_ref, b2_ref, ws_ref,
               bs_ref, d_ref, m_ref):
    ea = ea_ref[...]
    pre = (xs_ref[...] + xd_ref[...]
           + jnp.dot(ea, w1c_ref[...], preferred_element_type=jnp.float32))
    h = jnp.maximum(pre, 0.0)
    msg = jnp.maximum(
        jnp.dot(h, w2_ref[...], preferred_element_type=jnp.float32)
        + b2_ref[...], 0.0)
    d_ref[...] = ea + msg
    gate = jax.nn.sigmoid(
        jnp.sum(msg * ws_ref[...], axis=-1, keepdims=True) + bs_ref[...])
    m_ref[...] = msg * gate


def _edge(xs, xd, ea, w1c, w2, b2, ws, bs, d_prev, slab, e_tot, be=6400):
    e_s = ea.shape[0]
    nblk = e_s // be
    base_blk = slab * nblk
    blk = pl.BlockSpec((be, H), lambda i: (i, 0))
    pblk = pl.BlockSpec((be, H), lambda i: (i, 0))
    wblk = pl.BlockSpec((H, H), lambda i: (0, 0))
    rowblk = pl.BlockSpec((1, H), lambda i: (0, 0))
    in_specs = [pblk, pblk, blk, wblk, wblk, rowblk, rowblk,
                pl.BlockSpec((1, 1), lambda i: (0, 0))]
    inputs = [xs, xd, ea, w1c, w2, b2, ws, bs]
    io_alias = {}
    body = _edge_body
    if d_prev is not None:
        in_specs.append(pl.BlockSpec(memory_space=pl.ANY))
        inputs.append(d_prev)
        io_alias = {8: 0}
        body = _edge_body_alias
    return pl.pallas_call(
        body,
        grid=(nblk,),
        in_specs=in_specs,
        out_specs=[pl.BlockSpec((be, H), lambda i: (base_blk + i, 0)), blk],
        out_shape=[jax.ShapeDtypeStruct((e_tot, H), jnp.float32),
                   jax.ShapeDtypeStruct((e_s, H), jnp.float32)],
        input_output_aliases=io_alias,
    )(*inputs)


def _scatter(m, dst, zeros):
    e = m.shape[0]
    n = zeros.shape[1]
    per_w = e // _NW
    c_sz = 40
    rows_per_s = n // _NS
    mesh = plsc.VectorSubcoreMesh(core_axis_name="c", subcore_axis_name="s")

    scratch = ([pltpu.VMEM((c_sz,), jnp.int32)] * _NBUF
               + [pltpu.VMEM((c_sz, H), jnp.float32)] * _NBUF
               + [pltpu.VMEM_SHARED((n, H), jnp.float32)]
               + [pltpu.SemaphoreType.DMA] * (2 * _NBUF))

    @functools.partial(
        pl.kernel, mesh=mesh,
        out_type=jax.ShapeDtypeStruct((_NC, n, H), jnp.float32),
        scratch_types=scratch,
    )
    def k(m_hbm, dst_hbm, z_hbm, out_hbm, *scr):
        idx_v = scr[0:_NBUF]
        rows_v = scr[_NBUF:2 * _NBUF]
        acc = scr[2 * _NBUF]
        sem_l = scr[2 * _NBUF + 1:3 * _NBUF + 1]
        sem_s = scr[3 * _NBUF + 1:4 * _NBUF + 1]
        c = lax.axis_index("c")
        s = lax.axis_index("s")
        wid = s * _NC + c
        pltpu.sync_copy(z_hbm.at[c, pl.ds(s * rows_per_s, rows_per_s)],
                        acc.at[pl.ds(s * rows_per_s, rows_per_s)])
        plsc.subcore_barrier()
        base = wid * per_w

        def body(j, carry):
            offs = [base + (j * _NBUF + b) * c_sz for b in range(_NBUF)]
            for b in range(_NBUF):
                off = offs[b]

                @pl.when(j > 0)
                def _(b=b):
                    pltpu.make_async_copy(
                        rows_v[b], acc.at[idx_v[b]], sem_s[b]).wait()

                pltpu.async_copy(dst_hbm.at[pl.ds(off, c_sz)], idx_v[b], sem_l[b])
                pltpu.async_copy(m_hbm.at[pl.ds(off, c_sz)], rows_v[b], sem_l[b])
            for b in range(_NBUF):
                off = offs[b]
                pltpu.make_async_copy(
                    dst_hbm.at[pl.ds(off, c_sz)], idx_v[b], sem_l[b]).wait()
                pltpu.make_async_copy(
                    m_hbm.at[pl.ds(off, c_sz)], rows_v[b], sem_l[b]).wait()
                pltpu.async_copy(rows_v[b], acc.at[idx_v[b]], sem_s[b], add=True)
            return carry

        lax.fori_loop(0, per_w // (c_sz * _NBUF), body, 0)
        for b in range(_NBUF):
            pltpu.make_async_copy(rows_v[b], acc.at[idx_v[b]], sem_s[b]).wait()
        plsc.subcore_barrier()
        pltpu.sync_copy(acc.at[pl.ds(s * rows_per_s, rows_per_s)],
                        out_hbm.at[c, pl.ds(s * rows_per_s, rows_per_s)])

    return k(m, dst, zeros)


def _node_body(*refs):
    part_refs = refs[:-6]
    x_ref, wu1_ref, bu1_ref, wu2_ref, bu2_ref, o_ref = refs[-6:]
    x = x_ref[...]
    inp = x
    for p_ref in part_refs:
        p = p_ref[...]
        inp = inp + p[0] + p[1]
    h = jnp.maximum(
        jnp.dot(inp, wu1_ref[...], preferred_element_type=jnp.float32)
        + bu1_ref[...], 0.0)
    o_ref[...] = (jnp.dot(h, wu2_ref[...], preferred_element_type=jnp.float32)
                  + bu2_ref[...] + x)


def _node(parts, x, wu1, bu1, wu2, bu2, bn=1000):
    n = x.shape[0]
    pspec = pl.BlockSpec((_NC, bn, H), lambda i: (0, i, 0))
    return pl.pallas_call(
        _node_body,
        grid=(n // bn,),
        in_specs=[pspec] * len(parts) + [
            pl.BlockSpec((bn, H), lambda i: (i, 0)),
            pl.BlockSpec((H, H), lambda i: (0, 0)),
            pl.BlockSpec((1, H), lambda i: (0, 0)),
            pl.BlockSpec((H, H), lambda i: (0, 0)),
            pl.BlockSpec((1, H), lambda i: (0, 0)),
        ],
        out_specs=pl.BlockSpec((bn, H), lambda i: (i, 0)),
        out_shape=jax.ShapeDtypeStruct((n, H), jnp.float32),
    )(*parts, x, wu1, bu1, wu2, bu2)


def kernel(x, edge_index, edge_attr, W_msg1, b_msg1, W_msg2, b_msg2,
           W_soft, b_soft, W_upd1, b_upd1, W_upd2, b_upd2):
    n = x.shape[0]
    src = edge_index[0]
    dst = edge_index[1]
    w1a = W_msg1[:H]
    w1b = W_msg1[H:2 * H]
    w1c = W_msg1[2 * H:]
    b1 = b_msg1.reshape(1, H)
    b2 = b_msg2.reshape(1, H)
    ws = W_soft.reshape(1, H)
    bs = b_soft.reshape(1, 1)

    xa_bf, xb_bf = _pre(x, w1a, w1b, b1)
    n_pad = ((n + _NS * 8 - 1) // (_NS * 8)) * (_NS * 8)
    zeros = jnp.zeros((_NC, n_pad, H), jnp.float32)

    e_tot = edge_attr.shape[0]
    n_slabs = 1
    e_s = e_tot // n_slabs
    d_new = None
    parts = []
    for i in range(n_slabs):
        sl = slice(i * e_s, (i + 1) * e_s)
        xs, xd = _gather(xa_bf, xb_bf, src[sl], dst[sl])
        d_new, m = _edge(xs, xd, edge_attr[sl], w1c, W_msg2, b2, ws, bs,
                         d_new, i, e_tot)
        parts.append(_scatter(m, dst[sl], zeros))

    feat_out = _node(parts, x, W_upd1, b_upd1.reshape(1, H),
                     W_upd2, b_upd2.reshape(1, H))
    return feat_out, d_new


# speedup vs baseline: 1.2196x; 1.0243x over previous
import functools

import jax
import jax.numpy as jnp
from jax import lax
from jax.experimental import pallas as pl
from jax.experimental.pallas import tpu as pltpu
from jax.experimental.pallas import tpu_sc as plsc

H = 128
_NC = 2
_NS = 16
_NW = _NC * _NS


def _pre_body(x_ref, w1a_ref, w1b_ref, b1_ref, xa_ref, xb_ref):
    x = x_ref[...]
    xa = jnp.dot(x, w1a_ref[...], preferred_element_type=jnp.float32)
    xb = (jnp.dot(x, w1b_ref[...], preferred_element_type=jnp.float32)
          + b1_ref[...])
    xa_ref[...] = xa
    xb_ref[...] = xb


def _pre(x, w1a, w1b, b1, bn=1000):
    n = x.shape[0]
    return pl.pallas_call(
        _pre_body,
        grid=(n // bn,),
        in_specs=[
            pl.BlockSpec((bn, H), lambda i: (i, 0)),
            pl.BlockSpec((H, H), lambda i: (0, 0)),
            pl.BlockSpec((H, H), lambda i: (0, 0)),
            pl.BlockSpec((1, H), lambda i: (0, 0)),
        ],
        out_specs=[pl.BlockSpec((bn, H), lambda i: (i, 0))] * 2,
        out_shape=[jax.ShapeDtypeStruct((n, H), jnp.float32)] * 2,
    )(x, w1a, w1b, b1)


_NBUF = 5


def _gather(xa_bf, xb_bf, src, dst):
    e = src.shape[0]
    per_w = e // _NW
    c_sz = 80 if per_w % (80 * _NBUF) == 0 else 40
    n_groups = per_w // (c_sz * _NBUF)
    mesh = plsc.VectorSubcoreMesh(core_axis_name="c", subcore_axis_name="s")

    scratch = ([pltpu.VMEM((c_sz,), jnp.int32)] * (2 * _NBUF)
               + [pltpu.VMEM((c_sz, H), jnp.float32)] * (2 * _NBUF)
               + [pltpu.SemaphoreType.DMA] * (3 * _NBUF))

    @functools.partial(
        pl.kernel, mesh=mesh,
        out_type=[jax.ShapeDtypeStruct((e, H), jnp.float32)] * 2,
        scratch_types=scratch,
    )
    def k(xa_hbm, xb_hbm, src_hbm, dst_hbm, xs_hbm, xd_hbm, *scr):
        idx_s = scr[0:_NBUF]
        idx_d = scr[_NBUF:2 * _NBUF]
        rows_a = scr[2 * _NBUF:3 * _NBUF]
        rows_b = scr[3 * _NBUF:4 * _NBUF]
        sem_i = scr[4 * _NBUF:5 * _NBUF]
        sem_g = scr[5 * _NBUF:6 * _NBUF]
        sem_w = scr[6 * _NBUF:7 * _NBUF]
        wid = lax.axis_index("s") * _NC + lax.axis_index("c")
        base = wid * per_w

        def body(j, carry):
            offs = [base + (j * _NBUF + b) * c_sz for b in range(_NBUF)]
            for b in range(_NBUF):
                off = offs[b]

                @pl.when(j > 0)
                def _(off=off, b=b):
                    pltpu.make_async_copy(
                        rows_a[b], xs_hbm.at[pl.ds(off, c_sz)], sem_w[b]).wait()
                    pltpu.make_async_copy(
                        rows_b[b], xd_hbm.at[pl.ds(off, c_sz)], sem_w[b]).wait()

                pltpu.async_copy(src_hbm.at[pl.ds(off, c_sz)], idx_s[b], sem_i[b])
                pltpu.async_copy(dst_hbm.at[pl.ds(off, c_sz)], idx_d[b], sem_i[b])
            gathers = []
            for b in range(_NBUF):
                off = offs[b]
                pltpu.make_async_copy(
                    src_hbm.at[pl.ds(off, c_sz)], idx_s[b], sem_i[b]).wait()
                pltpu.make_async_copy(
                    dst_hbm.at[pl.ds(off, c_sz)], idx_d[b], sem_i[b]).wait()
                ga = pltpu.async_copy(xa_hbm.at[idx_s[b]], rows_a[b], sem_g[b])
                gb = pltpu.async_copy(xb_hbm.at[idx_d[b]], rows_b[b], sem_g[b])
                gathers.append((ga, gb))
            for b in range(_NBUF):
                off = offs[b]
                ga, gb = gathers[b]
                ga.wait()
                gb.wait()
                pltpu.async_copy(rows_a[b], xs_hbm.at[pl.ds(off, c_sz)], sem_w[b])
                pltpu.async_copy(rows_b[b], xd_hbm.at[pl.ds(off, c_sz)], sem_w[b])
            return carry

        lax.fori_loop(0, n_groups, body, 0)
        for b in range(_NBUF):
            pltpu.make_async_copy(
                rows_a[b], xs_hbm.at[pl.ds(base, c_sz)], sem_w[b]).wait()
            pltpu.make_async_copy(
                rows_b[b], xd_hbm.at[pl.ds(base, c_sz)], sem_w[b]).wait()

    return k(xa_bf, xb_bf, src, dst)


def _edge_body_alias(xs_ref, xd_ref, ea_ref, w1c_ref, w2_ref, b2_ref, ws_ref,
                     bs_ref, dprev_ref, d_ref, m_ref):
    del dprev_ref
    _edge_body(xs_ref, xd_ref, ea_ref, w1c_ref, w2_ref, b2_ref, ws_ref,
               bs_ref, d_ref, m_ref)


def _edge_body(xs_ref, xd_ref, ea_ref, w1c_ref, w2_ref, b2_ref, ws_ref,
               bs_ref, d_ref, m_ref):
    ea = ea_ref[...]
    pre = (xs_ref[...] + xd_ref[...]
           + jnp.dot(ea, w1c_ref[...], preferred_element_type=jnp.float32))
    h = jnp.maximum(pre, 0.0)
    msg = jnp.maximum(
        jnp.dot(h, w2_ref[...], preferred_element_type=jnp.float32)
        + b2_ref[...], 0.0)
    d_ref[...] = ea + msg
    gate = jax.nn.sigmoid(
        jnp.sum(msg * ws_ref[...], axis=-1, keepdims=True) + bs_ref[...])
    m_ref[...] = msg * gate


def _edge(xs, xd, ea, w1c, w2, b2, ws, bs, d_prev, slab, e_tot, be=8000):
    e_s = ea.shape[0]
    nblk = e_s // be
    base_blk = slab * nblk
    blk = pl.BlockSpec((be, H), lambda i: (i, 0))
    pblk = pl.BlockSpec((be, H), lambda i: (i, 0))
    wblk = pl.BlockSpec((H, H), lambda i: (0, 0))
    rowblk = pl.BlockSpec((1, H), lambda i: (0, 0))
    in_specs = [pblk, pblk, blk, wblk, wblk, rowblk, rowblk,
                pl.BlockSpec((1, 1), lambda i: (0, 0))]
    inputs = [xs, xd, ea, w1c, w2, b2, ws, bs]
    io_alias = {}
    body = _edge_body
    if d_prev is not None:
        in_specs.append(pl.BlockSpec(memory_space=pl.ANY))
        inputs.append(d_prev)
        io_alias = {8: 0}
        body = _edge_body_alias
    return pl.pallas_call(
        body,
        grid=(nblk,),
        in_specs=in_specs,
        out_specs=[pl.BlockSpec((be, H), lambda i: (base_blk + i, 0)), blk],
        out_shape=[jax.ShapeDtypeStruct((e_tot, H), jnp.float32),
                   jax.ShapeDtypeStruct((e_s, H), jnp.float32)],
        input_output_aliases=io_alias,
    )(*inputs)


def _scatter(m, dst, zeros):
    e = m.shape[0]
    n = zeros.shape[1]
    per_w = e // _NW
    c_sz = 40
    rows_per_s = n // _NS
    mesh = plsc.VectorSubcoreMesh(core_axis_name="c", subcore_axis_name="s")

    scratch = ([pltpu.VMEM((c_sz,), jnp.int32)] * _NBUF
               + [pltpu.VMEM((c_sz, H), jnp.float32)] * _NBUF
               + [pltpu.VMEM_SHARED((n, H), jnp.float32)]
               + [pltpu.SemaphoreType.DMA] * (2 * _NBUF))

    @functools.partial(
        pl.kernel, mesh=mesh,
        out_type=jax.ShapeDtypeStruct((_NC, n, H), jnp.float32),
        scratch_types=scratch,
    )
    def k(m_hbm, dst_hbm, z_hbm, out_hbm, *scr):
        idx_v = scr[0:_NBUF]
        rows_v = scr[_NBUF:2 * _NBUF]
        acc = scr[2 * _NBUF]
        sem_l = scr[2 * _NBUF + 1:3 * _NBUF + 1]
        sem_s = scr[3 * _NBUF + 1:4 * _NBUF + 1]
        c = lax.axis_index("c")
        s = lax.axis_index("s")
        wid = s * _NC + c
        pltpu.sync_copy(z_hbm.at[c, pl.ds(s * rows_per_s, rows_per_s)],
                        acc.at[pl.ds(s * rows_per_s, rows_per_s)])
        plsc.subcore_barrier()
        base = wid * per_w

        def body(j, carry):
            offs = [base + (j * _NBUF + b) * c_sz for b in range(_NBUF)]
            for b in range(_NBUF):
                off = offs[b]

                @pl.when(j > 0)
                def _(b=b):
                    pltpu.make_async_copy(
                        rows_v[b], acc.at[idx_v[b]], sem_s[b]).wait()

                pltpu.async_copy(dst_hbm.at[pl.ds(off, c_sz)], idx_v[b], sem_l[b])
                pltpu.async_copy(m_hbm.at[pl.ds(off, c_sz)], rows_v[b], sem_l[b])
            for b in range(_NBUF):
                off = offs[b]
                pltpu.make_async_copy(
                    dst_hbm.at[pl.ds(off, c_sz)], idx_v[b], sem_l[b]).wait()
                pltpu.make_async_copy(
                    m_hbm.at[pl.ds(off, c_sz)], rows_v[b], sem_l[b]).wait()
                pltpu.async_copy(rows_v[b], acc.at[idx_v[b]], sem_s[b], add=True)
            return carry

        lax.fori_loop(0, per_w // (c_sz * _NBUF), body, 0)
        for b in range(_NBUF):
            pltpu.make_async_copy(rows_v[b], acc.at[idx_v[b]], sem_s[b]).wait()
        plsc.subcore_barrier()
        pltpu.sync_copy(acc.at[pl.ds(s * rows_per_s, rows_per_s)],
                        out_hbm.at[c, pl.ds(s * rows_per_s, rows_per_s)])

    return k(m, dst, zeros)


def _node_body(*refs):
    part_refs = refs[:-6]
    x_ref, wu1_ref, bu1_ref, wu2_ref, bu2_ref, o_ref = refs[-6:]
    x = x_ref[...]
    inp = x
    for p_ref in part_refs:
        p = p_ref[...]
        inp = inp + p[0] + p[1]
    h = jnp.maximum(
        jnp.dot(inp, wu1_ref[...], preferred_element_type=jnp.float32)
        + bu1_ref[...], 0.0)
    o_ref[...] = (jnp.dot(h, wu2_ref[...], preferred_element_type=jnp.float32)
                  + bu2_ref[...] + x)


def _node(parts, x, wu1, bu1, wu2, bu2, bn=1000):
    n = x.shape[0]
    pspec = pl.BlockSpec((_NC, bn, H), lambda i: (0, i, 0))
    return pl.pallas_call(
        _node_body,
        grid=(n // bn,),
        in_specs=[pspec] * len(parts) + [
            pl.BlockSpec((bn, H), lambda i: (i, 0)),
            pl.BlockSpec((H, H), lambda i: (0, 0)),
            pl.BlockSpec((1, H), lambda i: (0, 0)),
            pl.BlockSpec((H, H), lambda i: (0, 0)),
            pl.BlockSpec((1, H), lambda i: (0, 0)),
        ],
        out_specs=pl.BlockSpec((bn, H), lambda i: (i, 0)),
        out_shape=jax.ShapeDtypeStruct((n, H), jnp.float32),
    )(*parts, x, wu1, bu1, wu2, bu2)


def kernel(x, edge_index, edge_attr, W_msg1, b_msg1, W_msg2, b_msg2,
           W_soft, b_soft, W_upd1, b_upd1, W_upd2, b_upd2):
    n = x.shape[0]
    src = edge_index[0]
    dst = edge_index[1]
    w1a = W_msg1[:H]
    w1b = W_msg1[H:2 * H]
    w1c = W_msg1[2 * H:]
    b1 = b_msg1.reshape(1, H)
    b2 = b_msg2.reshape(1, H)
    ws = W_soft.reshape(1, H)
    bs = b_soft.reshape(1, 1)

    xa_bf, xb_bf = _pre(x, w1a, w1b, b1)
    n_pad = ((n + _NS * 8 - 1) // (_NS * 8)) * (_NS * 8)
    zeros = jnp.zeros((_NC, n_pad, H), jnp.float32)

    e_tot = edge_attr.shape[0]
    n_slabs = 1
    e_s = e_tot // n_slabs
    d_new = None
    parts = []
    for i in range(n_slabs):
        sl = slice(i * e_s, (i + 1) * e_s)
        xs, xd = _gather(xa_bf, xb_bf, src[sl], dst[sl])
        d_new, m = _edge(xs, xd, edge_attr[sl], w1c, W_msg2, b2, ws, bs,
                         d_new, i, e_tot)
        parts.append(_scatter(m, dst[sl], zeros))

    feat_out = _node(parts, x, W_upd1, b_upd1.reshape(1, H),
                     W_upd2, b_upd2.reshape(1, H))
    return feat_out, d_new


# speedup vs baseline: 1.2220x; 1.0019x over previous
import functools

import jax
import jax.numpy as jnp
from jax import lax
from jax.experimental import pallas as pl
from jax.experimental.pallas import tpu as pltpu
from jax.experimental.pallas import tpu_sc as plsc

H = 128
_NC = 2
_NS = 16
_NW = _NC * _NS


def _pre_body(x_ref, w1a_ref, w1b_ref, b1_ref, xa_ref, xb_ref):
    x = x_ref[...]
    xa = jnp.dot(x, w1a_ref[...], preferred_element_type=jnp.float32)
    xb = (jnp.dot(x, w1b_ref[...], preferred_element_type=jnp.float32)
          + b1_ref[...])
    xa_ref[...] = xa
    xb_ref[...] = xb


def _pre(x, w1a, w1b, b1, bn=1000):
    n = x.shape[0]
    return pl.pallas_call(
        _pre_body,
        grid=(n // bn,),
        in_specs=[
            pl.BlockSpec((bn, H), lambda i: (i, 0)),
            pl.BlockSpec((H, H), lambda i: (0, 0)),
            pl.BlockSpec((H, H), lambda i: (0, 0)),
            pl.BlockSpec((1, H), lambda i: (0, 0)),
        ],
        out_specs=[pl.BlockSpec((bn, H), lambda i: (i, 0))] * 2,
        out_shape=[jax.ShapeDtypeStruct((n, H), jnp.float32)] * 2,
    )(x, w1a, w1b, b1)


_NBUF = 5


def _gather(xa_bf, xb_bf, src, dst):
    e = src.shape[0]
    per_w = e // _NW
    c_sz = 80 if per_w % (80 * _NBUF) == 0 else 40
    n_groups = per_w // (c_sz * _NBUF)
    mesh = plsc.VectorSubcoreMesh(core_axis_name="c", subcore_axis_name="s")

    scratch = ([pltpu.VMEM((c_sz,), jnp.int32)] * (2 * _NBUF)
               + [pltpu.VMEM((c_sz, H), jnp.float32)] * (2 * _NBUF)
               + [pltpu.SemaphoreType.DMA] * (3 * _NBUF))

    @functools.partial(
        pl.kernel, mesh=mesh,
        out_type=[jax.ShapeDtypeStruct((e, H), jnp.float32)] * 2,
        scratch_types=scratch,
    )
    def k(xa_hbm, xb_hbm, src_hbm, dst_hbm, xs_hbm, xd_hbm, *scr):
        idx_s = scr[0:_NBUF]
        idx_d = scr[_NBUF:2 * _NBUF]
        rows_a = scr[2 * _NBUF:3 * _NBUF]
        rows_b = scr[3 * _NBUF:4 * _NBUF]
        sem_i = scr[4 * _NBUF:5 * _NBUF]
        sem_g = scr[5 * _NBUF:6 * _NBUF]
        sem_w = scr[6 * _NBUF:7 * _NBUF]
        wid = lax.axis_index("s") * _NC + lax.axis_index("c")
        base = wid * per_w

        def body(j, carry):
            offs = [base + (j * _NBUF + b) * c_sz for b in range(_NBUF)]
            for b in range(_NBUF):
                off = offs[b]

                @pl.when(j > 0)
                def _(off=off, b=b):
                    pltpu.make_async_copy(
                        rows_a[b], xs_hbm.at[pl.ds(off, c_sz)], sem_w[b]).wait()
                    pltpu.make_async_copy(
                        rows_b[b], xd_hbm.at[pl.ds(off, c_sz)], sem_w[b]).wait()

                pltpu.async_copy(src_hbm.at[pl.ds(off, c_sz)], idx_s[b], sem_i[b])
                pltpu.async_copy(dst_hbm.at[pl.ds(off, c_sz)], idx_d[b], sem_i[b])
            gathers = []
            for b in range(_NBUF):
                off = offs[b]
                pltpu.make_async_copy(
                    src_hbm.at[pl.ds(off, c_sz)], idx_s[b], sem_i[b]).wait()
                pltpu.make_async_copy(
                    dst_hbm.at[pl.ds(off, c_sz)], idx_d[b], sem_i[b]).wait()
                ga = pltpu.async_copy(xa_hbm.at[idx_s[b]], rows_a[b], sem_g[b])
                gb = pltpu.async_copy(xb_hbm.at[idx_d[b]], rows_b[b], sem_g[b])
                gathers.append((ga, gb))
            for b in range(_NBUF):
                off = offs[b]
                ga, gb = gathers[b]
                ga.wait()
                gb.wait()
                pltpu.async_copy(rows_a[b], xs_hbm.at[pl.ds(off, c_sz)], sem_w[b])
                pltpu.async_copy(rows_b[b], xd_hbm.at[pl.ds(off, c_sz)], sem_w[b])
            return carry

        lax.fori_loop(0, n_groups, body, 0)
        for b in range(_NBUF):
            pltpu.make_async_copy(
                rows_a[b], xs_hbm.at[pl.ds(base, c_sz)], sem_w[b]).wait()
            pltpu.make_async_copy(
                rows_b[b], xd_hbm.at[pl.ds(base, c_sz)], sem_w[b]).wait()

    return k(xa_bf, xb_bf, src, dst)


def _edge_body_alias(xs_ref, xd_ref, ea_ref, w1c_ref, w2_ref, b2_ref, ws_ref,
                     bs_ref, dprev_ref, d_ref, m_ref):
    del dprev_ref
    _edge_body(xs_ref, xd_ref, ea_ref, w1c_ref, w2_ref, b2_ref, ws_ref,
               bs_ref, d_ref, m_ref)


def _edge_body(xs_ref, xd_ref, ea_ref, w1c_ref, w2_ref, b2_ref, ws_ref,
               bs_ref, d_ref, m_ref):
    ea = ea_ref[...]
    pre = (xs_ref[...] + xd_ref[...]
           + jnp.dot(ea, w1c_ref[...], preferred_element_type=jnp.float32))
    h = jnp.maximum(pre, 0.0)
    msg = jnp.maximum(
        jnp.dot(h, w2_ref[...], preferred_element_type=jnp.float32)
        + b2_ref[...], 0.0)
    d_ref[...] = ea + msg
    gate = jax.nn.sigmoid(
        jnp.sum(msg * ws_ref[...], axis=-1, keepdims=True) + bs_ref[...])
    m_ref[...] = msg * gate


def _edge(xs, xd, ea, w1c, w2, b2, ws, bs, d_prev, slab, e_tot, be=10000):
    e_s = ea.shape[0]
    nblk = e_s // be
    base_blk = slab * nblk
    blk = pl.BlockSpec((be, H), lambda i: (i, 0))
    pblk = pl.BlockSpec((be, H), lambda i: (i, 0))
    wblk = pl.BlockSpec((H, H), lambda i: (0, 0))
    rowblk = pl.BlockSpec((1, H), lambda i: (0, 0))
    in_specs = [pblk, pblk, blk, wblk, wblk, rowblk, rowblk,
                pl.BlockSpec((1, 1), lambda i: (0, 0))]
    inputs = [xs, xd, ea, w1c, w2, b2, ws, bs]
    io_alias = {}
    body = _edge_body
    if d_prev is not None:
        in_specs.append(pl.BlockSpec(memory_space=pl.ANY))
        inputs.append(d_prev)
        io_alias = {8: 0}
        body = _edge_body_alias
    return pl.pallas_call(
        body,
        grid=(nblk,),
        in_specs=in_specs,
        out_specs=[pl.BlockSpec((be, H), lambda i: (base_blk + i, 0)), blk],
        out_shape=[jax.ShapeDtypeStruct((e_tot, H), jnp.float32),
                   jax.ShapeDtypeStruct((e_s, H), jnp.float32)],
        input_output_aliases=io_alias,
    )(*inputs)


def _scatter(m, dst, zeros):
    e = m.shape[0]
    n = zeros.shape[1]
    per_w = e // _NW
    c_sz = 40
    rows_per_s = n // _NS
    mesh = plsc.VectorSubcoreMesh(core_axis_name="c", subcore_axis_name="s")

    scratch = ([pltpu.VMEM((c_sz,), jnp.int32)] * _NBUF
               + [pltpu.VMEM((c_sz, H), jnp.float32)] * _NBUF
               + [pltpu.VMEM_SHARED((n, H), jnp.float32)]
               + [pltpu.SemaphoreType.DMA] * (2 * _NBUF))

    @functools.partial(
        pl.kernel, mesh=mesh,
        out_type=jax.ShapeDtypeStruct((_NC, n, H), jnp.float32),
        scratch_types=scratch,
    )
    def k(m_hbm, dst_hbm, z_hbm, out_hbm, *scr):
        idx_v = scr[0:_NBUF]
        rows_v = scr[_NBUF:2 * _NBUF]
        acc = scr[2 * _NBUF]
        sem_l = scr[2 * _NBUF + 1:3 * _NBUF + 1]
        sem_s = scr[3 * _NBUF + 1:4 * _NBUF + 1]
        c = lax.axis_index("c")
        s = lax.axis_index("s")
        wid = s * _NC + c
        pltpu.sync_copy(z_hbm.at[c, pl.ds(s * rows_per_s, rows_per_s)],
                        acc.at[pl.ds(s * rows_per_s, rows_per_s)])
        plsc.subcore_barrier()
        base = wid * per_w

        def body(j, carry):
            offs = [base + (j * _NBUF + b) * c_sz for b in range(_NBUF)]
            for b in range(_NBUF):
                off = offs[b]

                @pl.when(j > 0)
                def _(b=b):
                    pltpu.make_async_copy(
                        rows_v[b], acc.at[idx_v[b]], sem_s[b]).wait()

                pltpu.async_copy(dst_hbm.at[pl.ds(off, c_sz)], idx_v[b], sem_l[b])
                pltpu.async_copy(m_hbm.at[pl.ds(off, c_sz)], rows_v[b], sem_l[b])
            for b in range(_NBUF):
                off = offs[b]
                pltpu.make_async_copy(
                    dst_hbm.at[pl.ds(off, c_sz)], idx_v[b], sem_l[b]).wait()
                pltpu.make_async_copy(
                    m_hbm.at[pl.ds(off, c_sz)], rows_v[b], sem_l[b]).wait()
                pltpu.async_copy(rows_v[b], acc.at[idx_v[b]], sem_s[b], add=True)
            return carry

        lax.fori_loop(0, per_w // (c_sz * _NBUF), body, 0)
        for b in range(_NBUF):
            pltpu.make_async_copy(rows_v[b], acc.at[idx_v[b]], sem_s[b]).wait()
        plsc.subcore_barrier()
        pltpu.sync_copy(acc.at[pl.ds(s * rows_per_s, rows_per_s)],
                        out_hbm.at[c, pl.ds(s * rows_per_s, rows_per_s)])

    return k(m, dst, zeros)


def _node_body(*refs):
    part_refs = refs[:-6]
    x_ref, wu1_ref, bu1_ref, wu2_ref, bu2_ref, o_ref = refs[-6:]
    x = x_ref[...]
    inp = x
    for p_ref in part_refs:
        p = p_ref[...]
        inp = inp + p[0] + p[1]
    h = jnp.maximum(
        jnp.dot(inp, wu1_ref[...], preferred_element_type=jnp.float32)
        + bu1_ref[...], 0.0)
    o_ref[...] = (jnp.dot(h, wu2_ref[...], preferred_element_type=jnp.float32)
                  + bu2_ref[...] + x)


def _node(parts, x, wu1, bu1, wu2, bu2, bn=1000):
    n = x.shape[0]
    pspec = pl.BlockSpec((_NC, bn, H), lambda i: (0, i, 0))
    return pl.pallas_call(
        _node_body,
        grid=(n // bn,),
        in_specs=[pspec] * len(parts) + [
            pl.BlockSpec((bn, H), lambda i: (i, 0)),
            pl.BlockSpec((H, H), lambda i: (0, 0)),
            pl.BlockSpec((1, H), lambda i: (0, 0)),
            pl.BlockSpec((H, H), lambda i: (0, 0)),
            pl.BlockSpec((1, H), lambda i: (0, 0)),
        ],
        out_specs=pl.BlockSpec((bn, H), lambda i: (i, 0)),
        out_shape=jax.ShapeDtypeStruct((n, H), jnp.float32),
    )(*parts, x, wu1, bu1, wu2, bu2)


def kernel(x, edge_index, edge_attr, W_msg1, b_msg1, W_msg2, b_msg2,
           W_soft, b_soft, W_upd1, b_upd1, W_upd2, b_upd2):
    n = x.shape[0]
    src = edge_index[0]
    dst = edge_index[1]
    w1a = W_msg1[:H]
    w1b = W_msg1[H:2 * H]
    w1c = W_msg1[2 * H:]
    b1 = b_msg1.reshape(1, H)
    b2 = b_msg2.reshape(1, H)
    ws = W_soft.reshape(1, H)
    bs = b_soft.reshape(1, 1)

    xa_bf, xb_bf = _pre(x, w1a, w1b, b1)
    n_pad = ((n + _NS * 8 - 1) // (_NS * 8)) * (_NS * 8)
    zeros = jnp.zeros((_NC, n_pad, H), jnp.float32)

    e_tot = edge_attr.shape[0]
    n_slabs = 1
    e_s = e_tot // n_slabs
    d_new = None
    parts = []
    for i in range(n_slabs):
        sl = slice(i * e_s, (i + 1) * e_s)
        xs, xd = _gather(xa_bf, xb_bf, src[sl], dst[sl])
        d_new, m = _edge(xs, xd, edge_attr[sl], w1c, W_msg2, b2, ws, bs,
                         d_new, i, e_tot)
        parts.append(_scatter(m, dst[sl], zeros))

    feat_out = _node(parts, x, W_upd1, b_upd1.reshape(1, H),
                     W_upd2, b_upd2.reshape(1, H))
    return feat_out, d_new
